# Initial kernel scaffold; baseline (speedup 1.0000x reference)
import jax
import jax.numpy as jnp
from jax.experimental import pallas as pl


def kernel(ly, lp, ry, emb_table, pos_table):
    raise NotImplementedError("write your pallas kernel here")


# speedup vs baseline: 1.1761x; 1.1761x over previous
import jax
import jax.numpy as jnp
from jax import lax
from jax.experimental import pallas as pl
from jax.experimental.pallas import tpu as pltpu
from jax.experimental.pallas import tpu_sc as plsc

_L = 200
_B = 1024
_M = 64
_N = _L * _B
_NG = _N - _B
_SUB = 128
_UNIT = 1024
_NSUB = _UNIT // _SUB
_NUNITS = _NG // _UNIT
_NTASK = 3
_TOT = _NTASK * _NUNITS
_NW = 32
_ZROWS = _B // _NW


def _body(ly_h, lp_h, ry_h, emb_h, pos_h, lo_h, po_h, ro_h,
          idx_v, rows_v, sem):
    c = lax.axis_index("c")
    s = lax.axis_index("s")
    w = s * 2 + c

    zvec = jnp.zeros((16,), jnp.float32)

    def _zrow(r, carry):
        for cc in range(_M // 16):
            rows_v[r, pl.ds(cc * 16, 16)] = zvec
        return carry

    lax.fori_loop(0, _ZROWS, _zrow, 0)
    zbase = w * _ZROWS
    for out_h in (lo_h, po_h, ro_h):
        pltpu.sync_copy(rows_v.at[pl.ds(0, _ZROWS)],
                        out_h.at[pl.ds(zbase, _ZROWS)])

    def _unit(idx_h, tab_h, out_h, irow, orow):
        pltpu.sync_copy(idx_h.at[pl.ds(irow, _NSUB)], idx_v)
        descs = [
            pltpu.async_copy(tab_h.at[idx_v.at[j]],
                             rows_v.at[pl.ds(j * _SUB, _SUB)], sem)
            for j in range(_NSUB)
        ]
        for d in descs:
            d.wait()
        pltpu.sync_copy(rows_v, out_h.at[pl.ds(orow, _UNIT)])

    nu = (_TOT // _NW) + jnp.where(w < (_TOT % _NW), 1, 0)

    def _step(i, carry):
        uid = w + i * _NW
        task = uid % _NTASK
        u = uid // _NTASK
        orow = _B + u * _UNIT
        irow = u * _NSUB

        @pl.when(task == 0)
        def _():
            _unit(ly_h, emb_h, lo_h, irow, orow)

        @pl.when(task == 1)
        def _():
            _unit(lp_h, pos_h, po_h, irow, orow)

        @pl.when(task == 2)
        def _():
            _unit(ry_h, emb_h, ro_h, _NSUB + irow, orow)

        return carry

    lax.fori_loop(0, nu, _step, 0)


@jax.jit
def kernel(ly, lp, ry, emb_table, pos_table):
    ly2 = ly.astype(jnp.int32).reshape(_N // _SUB, _SUB)
    lp2 = lp.astype(jnp.int32).reshape(_N // _SUB, _SUB)
    ry2 = ry.astype(jnp.int32).reshape(_N // _SUB, _SUB)

    mesh = plsc.VectorSubcoreMesh(core_axis_name="c", subcore_axis_name="s")
    out3 = (jax.ShapeDtypeStruct((_N, _M), jnp.float32),) * 3
    run = pl.kernel(
        _body,
        mesh=mesh,
        out_type=out3,
        scratch_types=[
            pltpu.VMEM((_NSUB, _SUB), jnp.int32),
            pltpu.VMEM((_UNIT, _M), jnp.float32),
            pltpu.SemaphoreType.DMA,
        ],
        compiler_params=pltpu.CompilerParams(use_tc_tiling_on_sc=False),
    )
    lo, po, ro = run(ly2, lp2, ry2, emb_table, pos_table)
    return (lo.reshape(_L, _B, _M),
            po.reshape(_L, _B, _M),
            ro.reshape(_L, _B, _M))
